# baseline (speedup 1.0000x reference)
import functools

import jax
import jax.numpy as jnp
from jax import lax
from jax.experimental import pallas as pl
from jax.experimental.pallas import tpu as pltpu
from jax.experimental.pallas import tpu_sc as plsc

_N_FIELDS = 26
_VOCAB = 100000
_DIM = 32
_LANES = 16

_NC = 2
_NS = 16
_NW = _NC * _NS

_CH = 100


def _sc_embed(xt, tables, *, rows_total):
    per_w = rows_total // _NW
    n_chunks = per_w // _CH
    mesh = plsc.VectorSubcoreMesh(core_axis_name="c", subcore_axis_name="s")

    @functools.partial(
        pl.kernel,
        mesh=mesh,
        compiler_params=pltpu.CompilerParams(use_tc_tiling_on_sc=False),
        out_type=jax.ShapeDtypeStruct((4096, 50, 1, _DIM), jnp.float32),
        scratch_types=[
            pltpu.VMEM((_N_FIELDS, _CH), jnp.int32),
            pltpu.VMEM((_N_FIELDS * _CH, _DIM), jnp.float32),
            pltpu.VMEM((_CH // 50, 50, 1, _DIM), jnp.float32),
            pltpu.SemaphoreType.DMA,
        ],
    )
    def k(xt_hbm, tab_hbm, out_hbm, idx_v, rows_v, out_v, sem):
        wid = lax.axis_index("s") * _NC + lax.axis_index("c")
        base = wid * per_w

        def chunk_body(c, carry):
            g = wid * n_chunks + c
            row0 = base + c * _CH
            pltpu.sync_copy(xt_hbm.at[:, g, :], idx_v)
            copies = [
                pltpu.async_copy(tab_hbm.at[i].at[idx_v.at[i]],
                                 rows_v.at[pl.ds(i * _CH, _CH)], sem)
                for i in range(_N_FIELDS)
            ]
            for cp in copies:
                cp.wait()

            def row_body(r, carry2):
                acc0 = rows_v[r, pl.ds(0, _LANES)]
                acc1 = rows_v[r, pl.ds(_LANES, _LANES)]
                for i in range(1, _N_FIELDS):
                    acc0 = acc0 + rows_v[i * _CH + r, pl.ds(0, _LANES)]
                    acc1 = acc1 + rows_v[i * _CH + r, pl.ds(_LANES, _LANES)]
                out_v[r // 50, r % 50, 0, pl.ds(0, _LANES)] = acc0
                out_v[r // 50, r % 50, 0, pl.ds(_LANES, _LANES)] = acc1
                return carry2

            lax.fori_loop(0, _CH, row_body, 0)
            pltpu.sync_copy(out_v, out_hbm.at[pl.ds(row0 // 50, _CH // 50)])
            return carry

        lax.fori_loop(0, n_chunks, chunk_body, 0)

    return k(xt, tables)


def kernel(x, tables):
    b, h, w, n = x.shape
    rows_total = b * h * w
    xt = x.reshape(rows_total, n).T.astype(jnp.int32)
    xt3 = xt.reshape(n, rows_total // _CH, _CH)
    return _sc_embed(xt3, tables, rows_total=rows_total)


# speedup vs baseline: 1.1800x; 1.1800x over previous
import functools

import jax
import jax.numpy as jnp
from jax import lax
from jax.experimental import pallas as pl
from jax.experimental.pallas import tpu as pltpu
from jax.experimental.pallas import tpu_sc as plsc

_N_FIELDS = 26
_VOCAB = 100000
_DIM = 32
_LANES = 16

_NC = 2
_NS = 16
_NW = _NC * _NS

_CH = 128


def _sc_embed(xt, tables, *, rows_total):
    per_w = rows_total // _NW
    n_chunks = per_w // _CH
    mesh = plsc.VectorSubcoreMesh(core_axis_name="c", subcore_axis_name="s")

    @functools.partial(
        pl.kernel,
        mesh=mesh,
        compiler_params=pltpu.CompilerParams(use_tc_tiling_on_sc=False),
        out_type=jax.ShapeDtypeStruct((rows_total, _DIM), jnp.float32),
        scratch_types=[
            pltpu.VMEM((_N_FIELDS, _CH), jnp.int32),
            pltpu.VMEM((_CH, _DIM), jnp.float32),
            pltpu.SemaphoreType.DMA,
        ],
    )
    def k(xt_hbm, tab_hbm, out_hbm, idx_v, acc_v, sem):
        wid = lax.axis_index("s") * _NC + lax.axis_index("c")
        base = wid * per_w
        zero16 = jnp.zeros((_LANES,), jnp.float32)

        def chunk_body(c, carry):
            row0 = base + c * _CH
            pltpu.sync_copy(xt_hbm.at[:, pl.ds(row0, _CH)], idx_v)

            def z(r, carry2):
                acc_v[r, pl.ds(0, _LANES)] = zero16
                acc_v[r, pl.ds(_LANES, _LANES)] = zero16
                return carry2

            lax.fori_loop(0, _CH, z, 0)

            copies = [
                pltpu.async_copy(tab_hbm.at[i].at[idx_v.at[i]],
                                 acc_v, sem, add=True)
                for i in range(_N_FIELDS)
            ]
            for cp in copies:
                cp.wait()

            pltpu.sync_copy(acc_v, out_hbm.at[pl.ds(row0, _CH)])
            return carry

        lax.fori_loop(0, n_chunks, chunk_body, 0)

    return k(xt, tables)


def kernel(x, tables):
    b, h, w, n = x.shape
    rows_total = b * h * w
    xt = x.reshape(rows_total, n).T.astype(jnp.int32)
    out = _sc_embed(xt, tables, rows_total=rows_total)
    return out.reshape(b, h, w, _DIM)


# speedup vs baseline: 1.2339x; 1.0457x over previous
import functools

import jax
import jax.numpy as jnp
from jax import lax
from jax.experimental import pallas as pl
from jax.experimental.pallas import tpu as pltpu
from jax.experimental.pallas import tpu_sc as plsc

_N_FIELDS = 26
_VOCAB = 100000
_DIM = 32
_LANES = 16

_NC = 2
_NS = 16
_NW = _NC * _NS

_CH = 128


def _sc_embed(xt, tables, *, rows_total):
    per_w = rows_total // _NW
    n_chunks = per_w // _CH
    mesh = plsc.VectorSubcoreMesh(core_axis_name="c", subcore_axis_name="s")

    @functools.partial(
        pl.kernel,
        mesh=mesh,
        compiler_params=pltpu.CompilerParams(use_tc_tiling_on_sc=False),
        out_type=jax.ShapeDtypeStruct((rows_total, _DIM), jnp.float32),
        scratch_types=[
            pltpu.VMEM((2, _N_FIELDS, _CH), jnp.int32),
            pltpu.VMEM((2, _CH, _DIM), jnp.float32),
            pltpu.SemaphoreType.DMA,
            pltpu.SemaphoreType.DMA,
        ],
    )
    def k(xt_hbm, tab_hbm, out_hbm, idx_v, acc_v, sem0, sem1):
        wid = lax.axis_index("s") * _NC + lax.axis_index("c")
        base = wid * per_w
        zero16 = jnp.zeros((_LANES,), jnp.float32)
        sems = (sem0, sem1)

        def issue(c):
            b = c % 2
            idx_b = idx_v.at[b]
            acc_b = acc_v.at[b]
            row0 = base + c * _CH
            pltpu.sync_copy(xt_hbm.at[:, pl.ds(row0, _CH)], idx_b)

            def z(r, carry):
                acc_b[r, pl.ds(0, _LANES)] = zero16
                acc_b[r, pl.ds(_LANES, _LANES)] = zero16
                return carry

            lax.fori_loop(0, _CH, z, 0)
            copies = [
                pltpu.async_copy(tab_hbm.at[i].at[idx_b.at[i]],
                                 acc_b, sems[b], add=True)
                for i in range(_N_FIELDS)
            ]
            return copies, row0, acc_b

        def drain(pending):
            copies, row0, acc_b = pending
            for cp in copies:
                cp.wait()
            pltpu.sync_copy(acc_b, out_hbm.at[pl.ds(row0, _CH)])

        pending = None
        for c in range(n_chunks):
            nxt = issue(c)
            if pending is not None:
                drain(pending)
            pending = nxt
        drain(pending)

    return k(xt, tables)


def kernel(x, tables):
    b, h, w, n = x.shape
    rows_total = b * h * w
    xt = x.reshape(rows_total, n).T.astype(jnp.int32)
    out = _sc_embed(xt, tables, rows_total=rows_total)
    return out.reshape(b, h, w, _DIM)


# speedup vs baseline: 1.2438x; 1.0080x over previous
import functools

import jax
import jax.numpy as jnp
from jax import lax
from jax.experimental import pallas as pl
from jax.experimental.pallas import tpu as pltpu
from jax.experimental.pallas import tpu_sc as plsc

_N_FIELDS = 26
_VOCAB = 100000
_DIM = 32
_LANES = 16

_NC = 2
_NS = 16
_NW = _NC * _NS

_CH = 256


def _sc_embed(xt, tables, *, rows_total):
    per_w = rows_total // _NW
    n_chunks = per_w // _CH
    mesh = plsc.VectorSubcoreMesh(core_axis_name="c", subcore_axis_name="s")

    @functools.partial(
        pl.kernel,
        mesh=mesh,
        compiler_params=pltpu.CompilerParams(use_tc_tiling_on_sc=False),
        out_type=jax.ShapeDtypeStruct((rows_total, _DIM), jnp.float32),
        scratch_types=[
            pltpu.VMEM((2, _N_FIELDS, _CH), jnp.int32),
            pltpu.VMEM((2, _CH, _DIM), jnp.float32),
            pltpu.SemaphoreType.DMA,
            pltpu.SemaphoreType.DMA,
        ],
    )
    def k(xt_hbm, tab_hbm, out_hbm, idx_v, acc_v, sem0, sem1):
        wid = lax.axis_index("s") * _NC + lax.axis_index("c")
        base = wid * per_w
        zero16 = jnp.zeros((_LANES,), jnp.float32)
        sems = (sem0, sem1)

        def issue(c):
            b = c % 2
            idx_b = idx_v.at[b]
            acc_b = acc_v.at[b]
            row0 = base + c * _CH
            pltpu.sync_copy(xt_hbm.at[:, pl.ds(row0, _CH)], idx_b)

            def z(r, carry):
                acc_b[r, pl.ds(0, _LANES)] = zero16
                acc_b[r, pl.ds(_LANES, _LANES)] = zero16
                return carry

            lax.fori_loop(0, _CH, z, 0)
            copies = [
                pltpu.async_copy(tab_hbm.at[i].at[idx_b.at[i]],
                                 acc_b, sems[b], add=True)
                for i in range(_N_FIELDS)
            ]
            return copies, row0, acc_b

        def drain(pending):
            copies, row0, acc_b = pending
            for cp in copies:
                cp.wait()
            pltpu.sync_copy(acc_b, out_hbm.at[pl.ds(row0, _CH)])

        pending = None
        for c in range(n_chunks):
            nxt = issue(c)
            if pending is not None:
                drain(pending)
            pending = nxt
        drain(pending)

    return k(xt, tables)


def kernel(x, tables):
    b, h, w, n = x.shape
    rows_total = b * h * w
    xt = x.reshape(rows_total, n).T.astype(jnp.int32)
    out = _sc_embed(xt, tables, rows_total=rows_total)
    return out.reshape(b, h, w, _DIM)
